# baseline (speedup 1.0000x reference)
import functools

import jax
import jax.numpy as jnp
from jax import lax
from jax.experimental import pallas as pl
from jax.experimental.pallas import tpu as pltpu
from jax.experimental.pallas import tpu_sc as plsc

BATCH = 16384
EMBED = 64
NTOK = 1000

NUM_CORES = 2
NUM_SUBCORES = 16
NW = NUM_CORES * NUM_SUBCORES
LANES = 128
WAVE = 4

NCHUNK = 2
CHUNK = BATCH // NCHUNK
BM = 2048
BLOCKS_PER_CHUNK = CHUNK // BM


def _gather_body(bpw, idx_hbm, tableT_hbm, outT_hbm, idx_v, slabs, rows_v,
                 sem_a, sem_b):
    wid = lax.axis_index("s") * NUM_CORES + lax.axis_index("c")
    base = pl.multiple_of(wid * bpw, bpw)
    pltpu.sync_copy(idx_hbm.at[pl.ds(base, bpw)], idx_v)

    rows16 = [lax.iota(jnp.int32, 16) + 16 * q for q in range(EMBED // 16)]

    def fire(ss, buf, sem):
        for k in range(WAVE):
            c = pl.multiple_of((ss[k] >> 7) << 7, LANES)
            pltpu.async_copy(
                tableT_hbm.at[:, pl.ds(c, LANES)], slabs.at[buf, k], sem)

    def wait_wave(buf, sem):
        for k in range(WAVE):
            pltpu.make_async_copy(
                tableT_hbm.at[:, pl.ds(0, LANES)], slabs.at[buf, k], sem
            ).wait()

    def select(ss, j, buf):
        for k in range(WAVE):
            lane = jnp.full((16,), ss[k] & (LANES - 1), jnp.int32)
            col = jnp.full((16,), j + k, jnp.int32)
            for q in range(EMBED // 16):
                vals = plsc.load_gather(slabs.at[buf, k], [rows16[q], lane])
                plsc.store_scatter(rows_v, [rows16[q], col], vals)

    def body(t, _):
        j0 = pl.multiple_of(t * 4 * WAVE, 4 * WAVE)
        vec = idx_v[pl.ds(j0, 16)]
        ss = [vec[k] for k in range(16)]
        w = [ss[WAVE * i:WAVE * (i + 1)] for i in range(4)]
        fire(w[0], 0, sem_a)
        fire(w[1], 1, sem_b)
        wait_wave(0, sem_a)
        select(w[0], j0, 0)
        fire(w[2], 0, sem_a)
        wait_wave(1, sem_b)
        select(w[1], j0 + WAVE, 1)
        fire(w[3], 1, sem_b)
        wait_wave(0, sem_a)
        select(w[2], j0 + 2 * WAVE, 0)
        wait_wave(1, sem_b)
        select(w[3], j0 + 3 * WAVE, 1)
        return ()

    lax.fori_loop(0, bpw // (4 * WAVE), body, ())
    pltpu.sync_copy(rows_v, outT_hbm.at[:, pl.ds(base, bpw)])


def _make_gather(nbatch):
    bpw = nbatch // NW
    return pl.kernel(
        functools.partial(_gather_body, bpw),
        out_type=jax.ShapeDtypeStruct((EMBED, nbatch), jnp.float32),
        mesh=plsc.VectorSubcoreMesh(core_axis_name="c", subcore_axis_name="s"),
        scratch_types=[
            pltpu.VMEM((bpw,), jnp.int32),
            pltpu.VMEM((2, WAVE, EMBED, LANES), jnp.float32),
            pltpu.VMEM((EMBED, bpw), jnp.float32),
            pltpu.SemaphoreType.DMA,
            pltpu.SemaphoreType.DMA,
        ],
        compiler_params=pltpu.CompilerParams(needs_layout_passes=False),
    )


_gather = _make_gather(CHUNK)


def _decode_body_first(ht_ref, d_ref, o_ref):
    o_ref[...] = jax.lax.dot_general(
        ht_ref[...], d_ref[...],
        dimension_numbers=(((0,), (0,)), ((), ())),
        preferred_element_type=jnp.float32,
    )


def _decode_body_chained(ht_ref, d_ref, prev_ref, o_ref):
    del prev_ref
    o_ref[...] = jax.lax.dot_general(
        ht_ref[...], d_ref[...],
        dimension_numbers=(((0,), (0,)), ((), ())),
        preferred_element_type=jnp.float32,
    )


def _make_decode(chunk_id):
    out_map = lambda i, c=chunk_id: (c * BLOCKS_PER_CHUNK + i, 0)
    in_specs = [
        pl.BlockSpec((EMBED, BM), lambda i: (0, i)),
        pl.BlockSpec((EMBED, NTOK), lambda i: (0, 0)),
    ]
    if chunk_id == 0:
        body = _decode_body_first
        aliases = {}
    else:
        body = _decode_body_chained
        in_specs = in_specs + [pl.BlockSpec(memory_space=pl.ANY)]
        aliases = {2: 0}
    return pl.pallas_call(
        body,
        grid=(BLOCKS_PER_CHUNK,),
        in_specs=in_specs,
        out_specs=pl.BlockSpec((BM, NTOK), out_map),
        out_shape=jax.ShapeDtypeStruct((BATCH, NTOK), jnp.float32),
        input_output_aliases=aliases,
        compiler_params=pltpu.CompilerParams(
            dimension_semantics=("arbitrary",),
        ),
    )


_decodes = [_make_decode(c) for c in range(NCHUNK)]


def kernel(x, embedding_weight, decoder_weight):
    xi = x.astype(jnp.int32)
    tableT = embedding_weight.T
    decT = decoder_weight.T
    hs = [_gather(xi[c * CHUNK:(c + 1) * CHUNK], tableT)
          for c in range(NCHUNK)]
    out = _decodes[0](hs[0], decT)
    for c in range(1, NCHUNK):
        out = _decodes[c](hs[c], decT, out)
    return out


# speedup vs baseline: 1.2017x; 1.2017x over previous
import functools

import jax
import jax.numpy as jnp
from jax import lax
from jax.experimental import pallas as pl
from jax.experimental.pallas import tpu as pltpu
from jax.experimental.pallas import tpu_sc as plsc

BATCH = 16384
EMBED = 64
NTOK = 1000

NUM_CORES = 2
NUM_SUBCORES = 16
NW = NUM_CORES * NUM_SUBCORES
LANES = 128
WAVE = 4

NCHUNK = 2
CHUNK = BATCH // NCHUNK
BM = 2048
BLOCKS_PER_CHUNK = CHUNK // BM


NSLOT = 8


def _gather_body(bpw, idx_hbm, tableT_hbm, outT_hbm, idx_v, slabs, rows_v,
                 *sems):
    wid = lax.axis_index("s") * NUM_CORES + lax.axis_index("c")
    base = pl.multiple_of(wid * bpw, bpw)
    pltpu.sync_copy(idx_hbm.at[pl.ds(base, bpw)], idx_v)

    rows16 = [lax.iota(jnp.int32, 16) + 16 * q for q in range(EMBED // 16)]

    def fire(s, slot):
        c = pl.multiple_of((s >> 7) << 7, LANES)
        pltpu.async_copy(
            tableT_hbm.at[:, pl.ds(c, LANES)], slabs.at[slot], sems[slot])

    def wait(slot):
        pltpu.make_async_copy(
            tableT_hbm.at[:, pl.ds(0, LANES)], slabs.at[slot], sems[slot]
        ).wait()

    def select(s, j, slot):
        lane = jnp.full((16,), s & (LANES - 1), jnp.int32)
        col = jnp.full((16,), j, jnp.int32)
        for q in range(EMBED // 16):
            vals = plsc.load_gather(slabs.at[slot], [rows16[q], lane])
            plsc.store_scatter(rows_v, [rows16[q], col], vals)

    nsteps = bpw // 16
    vec0 = idx_v[pl.ds(0, 16)]
    carry0 = [vec0[k] for k in range(16)]
    for k in range(NSLOT):
        fire(carry0[k], k)

    def body(t, carry):
        j0 = pl.multiple_of(t * 16, 16)
        for k in range(NSLOT):
            wait(k)
            select(carry[k], j0 + k, k)
            fire(carry[NSLOT + k], k)
        nb = pl.multiple_of(
            jnp.minimum((t + 1) * 16, bpw - 16).astype(jnp.int32), 16)
        vec_next = idx_v[pl.ds(nb, 16)]
        nxt = [vec_next[k] for k in range(16)]
        for k in range(NSLOT):
            wait(k)
            select(carry[NSLOT + k], j0 + NSLOT + k, k)

            @pl.when(t + 1 < nsteps)
            def _():
                fire(nxt[k], k)
        return nxt

    lax.fori_loop(0, nsteps, body, carry0)
    pltpu.sync_copy(rows_v, outT_hbm.at[:, pl.ds(base, bpw)])


def _make_gather(nbatch):
    bpw = nbatch // NW
    return pl.kernel(
        functools.partial(_gather_body, bpw),
        out_type=jax.ShapeDtypeStruct((EMBED, nbatch), jnp.float32),
        mesh=plsc.VectorSubcoreMesh(core_axis_name="c", subcore_axis_name="s"),
        scratch_types=[
            pltpu.VMEM((bpw,), jnp.int32),
            pltpu.VMEM((NSLOT, EMBED, LANES), jnp.float32),
            pltpu.VMEM((EMBED, bpw), jnp.float32),
        ] + [pltpu.SemaphoreType.DMA] * NSLOT,
        compiler_params=pltpu.CompilerParams(needs_layout_passes=False),
    )


_gather = _make_gather(CHUNK)


def _decode_body_first(ht_ref, d_ref, o_ref):
    o_ref[...] = jax.lax.dot_general(
        ht_ref[...], d_ref[...],
        dimension_numbers=(((0,), (0,)), ((), ())),
        preferred_element_type=jnp.float32,
    )


def _decode_body_chained(ht_ref, d_ref, prev_ref, o_ref):
    del prev_ref
    o_ref[...] = jax.lax.dot_general(
        ht_ref[...], d_ref[...],
        dimension_numbers=(((0,), (0,)), ((), ())),
        preferred_element_type=jnp.float32,
    )


def _make_decode(chunk_id):
    out_map = lambda i, c=chunk_id: (c * BLOCKS_PER_CHUNK + i, 0)
    in_specs = [
        pl.BlockSpec((EMBED, BM), lambda i: (0, i)),
        pl.BlockSpec((EMBED, NTOK), lambda i: (0, 0)),
    ]
    if chunk_id == 0:
        body = _decode_body_first
        aliases = {}
    else:
        body = _decode_body_chained
        in_specs = in_specs + [pl.BlockSpec(memory_space=pl.ANY)]
        aliases = {2: 0}
    return pl.pallas_call(
        body,
        grid=(BLOCKS_PER_CHUNK,),
        in_specs=in_specs,
        out_specs=pl.BlockSpec((BM, NTOK), out_map),
        out_shape=jax.ShapeDtypeStruct((BATCH, NTOK), jnp.float32),
        input_output_aliases=aliases,
        compiler_params=pltpu.CompilerParams(
            dimension_semantics=("arbitrary",),
        ),
    )


_decodes = [_make_decode(c) for c in range(NCHUNK)]


def kernel(x, embedding_weight, decoder_weight):
    xi = x.astype(jnp.int32)
    tableT = embedding_weight.T
    decT = decoder_weight.T
    hs = [_gather(xi[c * CHUNK:(c + 1) * CHUNK], tableT)
          for c in range(NCHUNK)]
    out = _decodes[0](hs[0], decT)
    for c in range(1, NCHUNK):
        out = _decodes[c](hs[c], decT, out)
    return out


# speedup vs baseline: 1.2297x; 1.0233x over previous
import functools

import jax
import jax.numpy as jnp
from jax import lax
from jax.experimental import pallas as pl
from jax.experimental.pallas import tpu as pltpu
from jax.experimental.pallas import tpu_sc as plsc

BATCH = 16384
EMBED = 64
NTOK = 1000

NUM_CORES = 2
NUM_SUBCORES = 16
NW = NUM_CORES * NUM_SUBCORES
LANES = 128
WAVE = 4

NCHUNK = 1
CHUNK = BATCH // NCHUNK
BM = 2048
BLOCKS_PER_CHUNK = CHUNK // BM


NSLOT = 8


def _gather_body(bpw, idx_hbm, tableT_hbm, outT_hbm, idx_v, slabs, rows_v,
                 *sems):
    wid = lax.axis_index("s") * NUM_CORES + lax.axis_index("c")
    base = pl.multiple_of(wid * bpw, bpw)
    pltpu.sync_copy(idx_hbm.at[pl.ds(base, bpw)], idx_v)

    rows16 = [lax.iota(jnp.int32, 16) + 16 * q for q in range(EMBED // 16)]

    def fire(s, slot):
        c = pl.multiple_of((s >> 7) << 7, LANES)
        pltpu.async_copy(
            tableT_hbm.at[:, pl.ds(c, LANES)], slabs.at[slot], sems[slot])

    def wait(slot):
        pltpu.make_async_copy(
            tableT_hbm.at[:, pl.ds(0, LANES)], slabs.at[slot], sems[slot]
        ).wait()

    def select(s, j, slot):
        lane = jnp.full((16,), s & (LANES - 1), jnp.int32)
        col = jnp.full((16,), j, jnp.int32)
        for q in range(EMBED // 16):
            vals = plsc.load_gather(slabs.at[slot], [rows16[q], lane])
            plsc.store_scatter(rows_v, [rows16[q], col], vals)

    nsteps = bpw // 16
    vec0 = idx_v[pl.ds(0, 16)]
    carry0 = [vec0[k] for k in range(16)]
    for k in range(NSLOT):
        fire(carry0[k], k)

    def body(t, carry):
        j0 = pl.multiple_of(t * 16, 16)
        for k in range(NSLOT):
            wait(k)
            select(carry[k], j0 + k, k)
            fire(carry[NSLOT + k], k)
        nb = pl.multiple_of(
            jnp.minimum((t + 1) * 16, bpw - 16).astype(jnp.int32), 16)
        vec_next = idx_v[pl.ds(nb, 16)]
        nxt = [vec_next[k] for k in range(16)]
        for k in range(NSLOT):
            wait(k)
            select(carry[NSLOT + k], j0 + NSLOT + k, k)

            @pl.when(t + 1 < nsteps)
            def _():
                fire(nxt[k], k)
        return nxt

    lax.fori_loop(0, nsteps, body, carry0)
    pltpu.sync_copy(rows_v, outT_hbm.at[:, pl.ds(base, bpw)])


def _make_gather(nbatch):
    bpw = nbatch // NW
    return pl.kernel(
        functools.partial(_gather_body, bpw),
        out_type=jax.ShapeDtypeStruct((EMBED, nbatch), jnp.float32),
        mesh=plsc.VectorSubcoreMesh(core_axis_name="c", subcore_axis_name="s"),
        scratch_types=[
            pltpu.VMEM((bpw,), jnp.int32),
            pltpu.VMEM((NSLOT, EMBED, LANES), jnp.float32),
            pltpu.VMEM((EMBED, bpw), jnp.float32),
        ] + [pltpu.SemaphoreType.DMA] * NSLOT,
        compiler_params=pltpu.CompilerParams(needs_layout_passes=False),
    )


_gather = _make_gather(CHUNK)


def _decode_body_first(ht_ref, d_ref, o_ref):
    o_ref[...] = jax.lax.dot_general(
        ht_ref[...], d_ref[...],
        dimension_numbers=(((0,), (0,)), ((), ())),
        preferred_element_type=jnp.float32,
    )


def _decode_body_chained(ht_ref, d_ref, prev_ref, o_ref):
    del prev_ref
    o_ref[...] = jax.lax.dot_general(
        ht_ref[...], d_ref[...],
        dimension_numbers=(((0,), (0,)), ((), ())),
        preferred_element_type=jnp.float32,
    )


def _make_decode(chunk_id):
    out_map = lambda i, c=chunk_id: (c * BLOCKS_PER_CHUNK + i, 0)
    in_specs = [
        pl.BlockSpec((EMBED, BM), lambda i: (0, i)),
        pl.BlockSpec((EMBED, NTOK), lambda i: (0, 0)),
    ]
    if chunk_id == 0:
        body = _decode_body_first
        aliases = {}
    else:
        body = _decode_body_chained
        in_specs = in_specs + [pl.BlockSpec(memory_space=pl.ANY)]
        aliases = {2: 0}
    return pl.pallas_call(
        body,
        grid=(BLOCKS_PER_CHUNK,),
        in_specs=in_specs,
        out_specs=pl.BlockSpec((BM, NTOK), out_map),
        out_shape=jax.ShapeDtypeStruct((BATCH, NTOK), jnp.float32),
        input_output_aliases=aliases,
        compiler_params=pltpu.CompilerParams(
            dimension_semantics=("arbitrary",),
        ),
    )


_decodes = [_make_decode(c) for c in range(NCHUNK)]


def kernel(x, embedding_weight, decoder_weight):
    hiddenT = _gather(x.astype(jnp.int32), embedding_weight.T)
    return _decodes[0](hiddenT, decoder_weight.T)


# speedup vs baseline: 1.5266x; 1.2414x over previous
import functools

import jax
import jax.numpy as jnp
from jax import lax
from jax.experimental import pallas as pl
from jax.experimental.pallas import tpu as pltpu
from jax.experimental.pallas import tpu_sc as plsc

BATCH = 16384
EMBED = 64
NTOK = 1000

NUM_CORES = 2
NUM_SUBCORES = 16
NW = NUM_CORES * NUM_SUBCORES
LANES = 128
WAVE = 4

NCHUNK = 1
CHUNK = BATCH // NCHUNK
BM = 2048
BLOCKS_PER_CHUNK = CHUNK // BM


NSLOT = 8


def _gather_body(bpw, idx_hbm, tableT_hbm, outT_hbm, idx_v, slabs, rows_v,
                 *sems):
    wid = lax.axis_index("s") * NUM_CORES + lax.axis_index("c")
    base = pl.multiple_of(wid * bpw, bpw)
    pltpu.sync_copy(idx_hbm.at[pl.ds(base, bpw)], idx_v)

    rows16 = [lax.iota(jnp.int32, 16) + 16 * q for q in range(EMBED // 16)]

    def fire(s, slot):
        c = pl.multiple_of((s >> 7) << 7, LANES)
        pltpu.async_copy(
            tableT_hbm.at[:, pl.ds(c, LANES)], slabs.at[slot], sems[slot])

    def wait(slot):
        pltpu.make_async_copy(
            tableT_hbm.at[:, pl.ds(0, LANES)], slabs.at[slot], sems[slot]
        ).wait()

    def select(s, j, slot):
        lane = jnp.full((16,), s & (LANES - 1), jnp.int32)
        col = jnp.full((16,), j, jnp.int32)
        for q in range(EMBED // 16):
            vals = plsc.load_gather(slabs.at[slot], [rows16[q], lane])
            plsc.store_scatter(rows_v, [rows16[q], col], vals)

    nsteps = bpw // 16
    vec0 = idx_v[pl.ds(0, 16)]
    carry0 = [vec0[k] for k in range(16)]
    for k in range(NSLOT):
        fire(carry0[k], k)

    def body(t, carry):
        j0 = pl.multiple_of(t * 16, 16)
        for k in range(NSLOT):
            wait(k)
            select(carry[k], j0 + k, k)
            fire(carry[NSLOT + k], k)
        nb = pl.multiple_of(
            jnp.minimum((t + 1) * 16, bpw - 16).astype(jnp.int32), 16)
        vec_next = idx_v[pl.ds(nb, 16)]
        nxt = [vec_next[k] for k in range(16)]
        for k in range(NSLOT):
            wait(k)
            select(carry[NSLOT + k], j0 + NSLOT + k, k)

            @pl.when(t + 1 < nsteps)
            def _():
                fire(nxt[k], k)
        return nxt

    lax.fori_loop(0, nsteps, body, carry0)
    pltpu.sync_copy(rows_v, outT_hbm.at[:, pl.ds(base, bpw)])


def _make_gather(nbatch):
    bpw = nbatch // NW
    return pl.kernel(
        functools.partial(_gather_body, bpw),
        out_type=jax.ShapeDtypeStruct((EMBED, nbatch), jnp.float32),
        mesh=plsc.VectorSubcoreMesh(core_axis_name="c", subcore_axis_name="s"),
        scratch_types=[
            pltpu.VMEM((bpw,), jnp.int32),
            pltpu.VMEM((NSLOT, EMBED, LANES), jnp.float32),
            pltpu.VMEM((EMBED, bpw), jnp.float32),
        ] + [pltpu.SemaphoreType.DMA] * NSLOT,
        compiler_params=pltpu.CompilerParams(needs_layout_passes=False),
    )


_gather = _make_gather(CHUNK)


def _decode_body(d_ref, ht_ref, o_ref):
    o_ref[...] = jax.lax.dot_general(
        d_ref[...], ht_ref[...],
        dimension_numbers=(((0,), (0,)), ((), ())),
        preferred_element_type=jnp.float32,
    )


_decode = pl.pallas_call(
    _decode_body,
    grid=(BATCH // BM,),
    in_specs=[
        pl.BlockSpec((EMBED, NTOK), lambda i: (0, 0)),
        pl.BlockSpec((EMBED, BM), lambda i: (0, i)),
    ],
    out_specs=pl.BlockSpec((NTOK, BM), lambda i: (0, i)),
    out_shape=jax.ShapeDtypeStruct((NTOK, BATCH), jnp.float32),
    compiler_params=pltpu.CompilerParams(
        dimension_semantics=("arbitrary",),
    ),
)


def kernel(x, embedding_weight, decoder_weight):
    hiddenT = _gather(x.astype(jnp.int32), embedding_weight.T)
    outT = _decode(decoder_weight.T, hiddenT)
    return outT.T
